# baseline (speedup 1.0000x reference)
import jax
import jax.numpy as jnp
from jax.experimental import pallas as pl
from jax.experimental.pallas import tpu as pltpu

TAU_ = 0.07
N_X = 1024
N_Y = 100000
N_YP = 100352
D = 16
K = 16
B = 3584
NB = N_YP // B
BIGF = 3.0e38
NEG = float("-inf")


def _topk_kernel(x_ref, yt_ref, out_ref, rv_ref, rif_ref, av_ref,
                 pm_ref, zsum_ref):
    j = pl.program_id(0)
    s = jnp.dot(x_ref[:], yt_ref[:], preferred_element_type=jnp.float32)
    lane16 = jax.lax.broadcasted_iota(jnp.int32, (N_X, K), 1)
    colf = (jax.lax.broadcasted_iota(jnp.int32, (N_X, B), 1)
            .astype(jnp.float32) + jnp.float32(j * B))
    sm = jnp.where(colf < N_Y, s, NEG)

    @pl.when(j == 0)
    def _init():
        rv_ref[:] = jnp.full((N_X, 128), NEG, jnp.float32)
        rif_ref[:] = jnp.full((N_X, 128), BIGF, jnp.float32)
        av_ref[:] = s[:, 0:K]

    lane128 = jax.lax.broadcasted_iota(jnp.int32, (N_X, 128), 1)
    nv = jnp.full((N_X, 128), NEG, jnp.float32)
    nif = jnp.full((N_X, 128), BIGF, jnp.float32)
    NV = (128 + B) // 128
    vals = [rv_ref[:]] + [sm[:, k * 128:(k + 1) * 128] for k in range(B // 128)]
    cols = [rif_ref[:]] + [colf[:, k * 128:(k + 1) * 128] for k in range(B // 128)]
    m0 = None
    for i in range(K):
        fv, fi_ = list(vals), list(cols)
        while len(fv) > 1:
            nfv, nfi = [], []
            for k in range(0, len(fv) - 1, 2):
                left = fv[k] >= fv[k + 1]
                nfv.append(jnp.where(left, fv[k], fv[k + 1]))
                nfi.append(jnp.where(left, fi_[k], fi_[k + 1]))
            if len(fv) % 2:
                nfv.append(fv[-1])
                nfi.append(fi_[-1])
            fv, fi_ = nfv, nfi
        F, FI = fv[0], fi_[0]
        m = jnp.max(F, axis=1, keepdims=True)
        gi = jnp.min(jnp.where(F >= m, FI, BIGF), axis=1, keepdims=True)
        nv = jnp.where(lane128 == i, m, nv)
        nif = jnp.where(lane128 == i, gi, nif)
        vals = [jnp.where(c == gi, NEG, v) for v, c in zip(vals, cols)]
        if m0 is None:
            m0 = m
    rv_ref[:] = nv
    rif_ref[:] = nif

    a2 = m0 / TAU_
    bsum = jnp.sum(jnp.exp(sm / TAU_ - a2), axis=1, keepdims=True)

    @pl.when(j == 0)
    def _zinit():
        zsum_ref[:] = bsum

    @pl.when(j != 0)
    def _zacc():
        zsum_ref[:] = zsum_ref[:] * jnp.exp(pm_ref[:] / TAU_ - a2) + bsum

    pm_ref[:] = m0

    @pl.when(j == NB - 1)
    def _merge():
        lse = jnp.log(zsum_ref[:]) + a2
        lane16f = lane16.astype(jnp.float32)
        cv = jnp.concatenate([nv[:, 0:K], av_ref[:]], axis=1)
        ci = jnp.concatenate([nif[:, 0:K], lane16f], axis=1)
        q = jnp.exp(cv / TAU_ - lse)
        res = jnp.zeros((N_X, K), jnp.float32)
        for i in range(K):
            m = jnp.max(q, axis=1, keepdims=True)
            sel = q >= m
            gi = jnp.min(jnp.where(sel, ci, BIGF), axis=1, keepdims=True)
            res = jnp.where(lane16 == i, gi, res)
            q = jnp.where(sel & (ci == gi), -1.0, q)
        out_ref[:] = res.astype(jnp.int32)


@jax.jit
def kernel(X, Y):
    yt = jnp.concatenate(
        [Y.T, jnp.zeros((D, N_YP - N_Y), jnp.float32)], axis=1)
    return pl.pallas_call(
        _topk_kernel,
        grid=(NB,),
        in_specs=[
            pl.BlockSpec((N_X, D), lambda j: (0, 0)),
            pl.BlockSpec((D, B), lambda j: (0, j)),
        ],
        out_specs=pl.BlockSpec((N_X, K), lambda j: (0, 0)),
        out_shape=jax.ShapeDtypeStruct((N_X, K), jnp.int32),
        scratch_shapes=[
            pltpu.VMEM((N_X, 128), jnp.float32),
            pltpu.VMEM((N_X, 128), jnp.float32),
            pltpu.VMEM((N_X, K), jnp.float32),
            pltpu.VMEM((N_X, 1), jnp.float32),
            pltpu.VMEM((N_X, 1), jnp.float32),
        ],
        compiler_params=pltpu.CompilerParams(
            dimension_semantics=("arbitrary",)),
    )(X, yt)


# speedup vs baseline: 1.1455x; 1.1455x over previous
import jax
import jax.numpy as jnp
from jax.experimental import pallas as pl
from jax.experimental.pallas import tpu as pltpu

TAU_ = 0.07
N_X = 1024
N_Y = 100000
N_YP = 100352
D = 16
K = 16
B = 3584
NB = N_YP // B
BIGF = 3.0e38
NEG = float("-inf")


def _topk_kernel(x_ref, yt_ref, out_ref, rv_ref, rif_ref, av_ref,
                 pm_ref, zsum_ref):
    j = pl.program_id(0)
    s = jnp.dot(x_ref[:], yt_ref[:], preferred_element_type=jnp.float32)
    lane16 = jax.lax.broadcasted_iota(jnp.int32, (N_X, K), 1)
    colf = (jax.lax.broadcasted_iota(jnp.int32, (N_X, B), 1)
            .astype(jnp.float32) + jnp.float32(j * B))
    sm = jnp.where(colf < N_Y, s, NEG)

    @pl.when(j == 0)
    def _init():
        rv_ref[:] = jnp.full((N_X, 128), NEG, jnp.float32)
        rif_ref[:] = jnp.full((N_X, 128), BIGF, jnp.float32)
        av_ref[:] = s[:, 0:K]

    smc = jnp.concatenate([rv_ref[:], sm], axis=1)
    cic = jnp.concatenate([rif_ref[:], colf], axis=1)
    lane128 = jax.lax.broadcasted_iota(jnp.int32, (N_X, 128), 1)
    nv = jnp.full((N_X, 128), NEG, jnp.float32)
    nif = jnp.full((N_X, 128), BIGF, jnp.float32)
    m0 = None
    for i in range(K):
        m = jnp.max(smc, axis=1, keepdims=True)
        gi = jnp.min(jnp.where(smc >= m, cic, BIGF), axis=1, keepdims=True)
        nv = jnp.where(lane128 == i, m, nv)
        nif = jnp.where(lane128 == i, gi, nif)
        smc = jnp.where(cic == gi, NEG, smc)
        if m0 is None:
            m0 = m
    rv_ref[:] = nv
    rif_ref[:] = nif

    a2 = m0 / TAU_
    bsum = jnp.sum(jnp.exp(sm / TAU_ - a2), axis=1, keepdims=True)

    @pl.when(j == 0)
    def _zinit():
        zsum_ref[:] = bsum

    @pl.when(j != 0)
    def _zacc():
        zsum_ref[:] = zsum_ref[:] * jnp.exp(pm_ref[:] / TAU_ - a2) + bsum

    pm_ref[:] = m0

    @pl.when(j == NB - 1)
    def _merge():
        lse = jnp.log(zsum_ref[:]) + a2
        lane16f = lane16.astype(jnp.float32)
        cv = jnp.concatenate([nv[:, 0:K], av_ref[:]], axis=1)
        ci = jnp.concatenate([nif[:, 0:K], lane16f], axis=1)
        q = jnp.exp(cv / TAU_ - lse)
        res = jnp.zeros((N_X, K), jnp.float32)
        for i in range(K):
            m = jnp.max(q, axis=1, keepdims=True)
            sel = q >= m
            gi = jnp.min(jnp.where(sel, ci, BIGF), axis=1, keepdims=True)
            res = jnp.where(lane16 == i, gi, res)
            q = jnp.where(sel & (ci == gi), -1.0, q)
        out_ref[:] = res.astype(jnp.int32)


@jax.jit
def kernel(X, Y):
    yt = jnp.concatenate(
        [Y.T, jnp.zeros((D, N_YP - N_Y), jnp.float32)], axis=1)
    return pl.pallas_call(
        _topk_kernel,
        grid=(NB,),
        in_specs=[
            pl.BlockSpec((N_X, D), lambda j: (0, 0)),
            pl.BlockSpec((D, B), lambda j: (0, j)),
        ],
        out_specs=pl.BlockSpec((N_X, K), lambda j: (0, 0)),
        out_shape=jax.ShapeDtypeStruct((N_X, K), jnp.int32),
        scratch_shapes=[
            pltpu.VMEM((N_X, 128), jnp.float32),
            pltpu.VMEM((N_X, 128), jnp.float32),
            pltpu.VMEM((N_X, K), jnp.float32),
            pltpu.VMEM((N_X, 1), jnp.float32),
            pltpu.VMEM((N_X, 1), jnp.float32),
        ],
        compiler_params=pltpu.CompilerParams(
            dimension_semantics=("arbitrary",)),
    )(X, yt)
